# baseline (speedup 1.0000x reference)
import functools

import jax
import jax.numpy as jnp
from jax import lax
from jax.experimental import pallas as pl
from jax.experimental.pallas import tpu as pltpu
from jax.experimental.pallas import tpu_sc as plsc

V = 32000
IGNORE = 0
SM = 0.1 / (V - 2)
CONF = 0.9

N_ROWS = 2048
ROW_BLK = 64
NBUF = 4
LANES = 16
ROWS_PER_TILE = N_ROWS // 32


def _tc_outer(x_hbm, t_ref, a_ref, c_ref):
    a_acc = [jnp.zeros((1, 1), jnp.float32)]

    def inner(idxs, x_ref):
        i = idxs[0]
        x = x_ref[...]
        m = jnp.max(x, axis=1, keepdims=True)
        se = jnp.sum(jnp.exp(x - m), axis=1, keepdims=True)
        lse = m + jnp.log(se)
        s = jnp.sum(x, axis=1, keepdims=True)
        x0 = x[:, 0:1]
        valid = t_ref[pl.ds(i * ROW_BLK, ROW_BLK), :] != IGNORE
        per_row = lse - SM * s + SM * x0

        @pl.when(i == 0)
        def _():
            a_ref[...] = jnp.zeros((1, 1), jnp.float32)
            c_ref[...] = jnp.zeros((1, 1), jnp.float32)

        a_ref[...] += jnp.sum(jnp.where(valid, per_row, 0.0),
                              axis=(0, 1), keepdims=True)
        c_ref[...] += jnp.sum(jnp.where(valid, 1.0, 0.0),
                              axis=(0, 1), keepdims=True)

    pltpu.emit_pipeline(
        inner,
        grid=(N_ROWS // ROW_BLK,),
        in_specs=[
            pl.BlockSpec((ROW_BLK, V), lambda i: (i, 0),
                         pipeline_mode=pl.Buffered(buffer_count=NBUF,
                                                   use_lookahead=False)),
        ],
        _explicit_indices=True,
    )(x_hbm)


def _tc_stats(out2d, tgt2d):
    return pl.pallas_call(
        _tc_outer,
        in_specs=[
            pl.BlockSpec(memory_space=pl.ANY),
            pl.BlockSpec(memory_space=pltpu.MemorySpace.VMEM),
        ],
        out_specs=[
            pl.BlockSpec(memory_space=pltpu.MemorySpace.VMEM),
            pl.BlockSpec(memory_space=pltpu.MemorySpace.VMEM),
        ],
        out_shape=[
            jax.ShapeDtypeStruct((1, 1), jnp.float32),
            jax.ShapeDtypeStruct((1, 1), jnp.float32),
        ],
    )(out2d, tgt2d)


def _sc_body(tab, tgt, out, tgt_v, idx_v, rows_v, acc_v, shared, all_v, sem):
    c = lax.axis_index("c")
    s = lax.axis_index("s")
    base = (c * 16 + s) * ROWS_PER_TILE
    pltpu.sync_copy(tgt.at[pl.ds(base, ROWS_PER_TILE)], tgt_v)
    iota = lax.iota(jnp.int32, LANES)
    for g in range(ROWS_PER_TILE // LANES):
        t = tgt_v[pl.ds(g * LANES, LANES)]
        rows = base + g * LANES + iota
        idx_v[pl.ds(g * LANES, LANES)] = rows * V + t
    pltpu.async_copy(tab.at[idx_v], rows_v, sem).wait()
    acc = jnp.zeros((LANES,), jnp.float32)
    zero = jnp.zeros((LANES,), jnp.float32)
    for g in range(ROWS_PER_TILE // LANES):
        t = tgt_v[pl.ds(g * LANES, LANES)]
        val = rows_v[pl.ds(g * LANES, LANES)]
        acc = acc + jnp.where(t != IGNORE, val, zero)
    acc_v[...] = acc
    pltpu.sync_copy(acc_v, shared.at[s])
    plsc.subcore_barrier()

    @pl.when(s == 0)
    def _():
        pltpu.sync_copy(shared, all_v)
        tot = jnp.zeros((LANES,), jnp.float32)
        for w in range(16):
            tot = tot + all_v[w]
        acc_v[...] = tot
        pltpu.sync_copy(acc_v, out.at[c])


_sc_gather = functools.partial(
    pl.kernel,
    mesh=plsc.VectorSubcoreMesh(core_axis_name="c", subcore_axis_name="s"),
    out_type=jax.ShapeDtypeStruct((2, LANES), jnp.float32),
    scratch_types=[
        pltpu.VMEM((ROWS_PER_TILE,), jnp.int32),
        pltpu.VMEM((ROWS_PER_TILE,), jnp.int32),
        pltpu.VMEM((ROWS_PER_TILE,), jnp.float32),
        pltpu.VMEM((LANES,), jnp.float32),
        pltpu.VMEM_SHARED((16, LANES), jnp.float32),
        pltpu.VMEM((16, LANES), jnp.float32),
        pltpu.SemaphoreType.DMA,
    ],
)(_sc_body)


def kernel(output, target):
    out2d = output.reshape(N_ROWS, V)
    tgt = target.reshape(N_ROWS).astype(jnp.int32)
    tab = out2d.reshape(N_ROWS * V)
    part = _sc_gather(tab, tgt)
    a, cnt = _tc_stats(out2d, tgt.reshape(N_ROWS, 1))
    t_sum = jnp.sum(part)
    return (a[0, 0] - (CONF - SM) * t_sum) / cnt[0, 0]


# speedup vs baseline: 1.0196x; 1.0196x over previous
import functools

import jax
import jax.numpy as jnp
from jax import lax
from jax.experimental import pallas as pl
from jax.experimental.pallas import tpu as pltpu
from jax.experimental.pallas import tpu_sc as plsc

V = 32000
IGNORE = 0
SM = 0.1 / (V - 2)
CONF = 0.9

N_ROWS = 2048
ROW_BLK = 128
NBUF = 3
LANES = 16
ROWS_PER_TILE = N_ROWS // 32


def _tc_outer(x_hbm, t_ref, a_ref, c_ref):
    a_acc = [jnp.zeros((1, 1), jnp.float32)]

    def inner(idxs, x_ref):
        i = idxs[0]
        x = x_ref[...]
        m = jnp.max(x, axis=1, keepdims=True)
        se = jnp.sum(jnp.exp(x - m), axis=1, keepdims=True)
        lse = m + jnp.log(se)
        s = jnp.sum(x, axis=1, keepdims=True)
        x0 = x[:, 0:1]
        valid = t_ref[pl.ds(i * ROW_BLK, ROW_BLK), :] != IGNORE
        per_row = lse - SM * s + SM * x0

        @pl.when(i == 0)
        def _():
            a_ref[...] = jnp.zeros((1, 1), jnp.float32)
            c_ref[...] = jnp.zeros((1, 1), jnp.float32)

        a_ref[...] += jnp.sum(jnp.where(valid, per_row, 0.0),
                              axis=(0, 1), keepdims=True)
        c_ref[...] += jnp.sum(jnp.where(valid, 1.0, 0.0),
                              axis=(0, 1), keepdims=True)

    pltpu.emit_pipeline(
        inner,
        grid=(N_ROWS // ROW_BLK,),
        in_specs=[
            pl.BlockSpec((ROW_BLK, V), lambda i: (i, 0),
                         pipeline_mode=pl.Buffered(buffer_count=NBUF,
                                                   use_lookahead=False)),
        ],
        _explicit_indices=True,
    )(x_hbm)


def _tc_stats(out2d, tgt2d):
    return pl.pallas_call(
        _tc_outer,
        in_specs=[
            pl.BlockSpec(memory_space=pl.ANY),
            pl.BlockSpec(memory_space=pltpu.MemorySpace.VMEM),
        ],
        out_specs=[
            pl.BlockSpec(memory_space=pltpu.MemorySpace.VMEM),
            pl.BlockSpec(memory_space=pltpu.MemorySpace.VMEM),
        ],
        out_shape=[
            jax.ShapeDtypeStruct((1, 1), jnp.float32),
            jax.ShapeDtypeStruct((1, 1), jnp.float32),
        ],
    )(out2d, tgt2d)


def _sc_body(tab, tgt, out, tgt_v, idx_v, rows_v, acc_v, shared, all_v, sem):
    c = lax.axis_index("c")
    s = lax.axis_index("s")
    base = (c * 16 + s) * ROWS_PER_TILE
    pltpu.sync_copy(tgt.at[pl.ds(base, ROWS_PER_TILE)], tgt_v)
    iota = lax.iota(jnp.int32, LANES)
    for g in range(ROWS_PER_TILE // LANES):
        t = tgt_v[pl.ds(g * LANES, LANES)]
        rows = base + g * LANES + iota
        idx_v[pl.ds(g * LANES, LANES)] = rows * V + t
    pltpu.async_copy(tab.at[idx_v], rows_v, sem).wait()
    acc = jnp.zeros((LANES,), jnp.float32)
    zero = jnp.zeros((LANES,), jnp.float32)
    for g in range(ROWS_PER_TILE // LANES):
        t = tgt_v[pl.ds(g * LANES, LANES)]
        val = rows_v[pl.ds(g * LANES, LANES)]
        acc = acc + jnp.where(t != IGNORE, val, zero)
    acc_v[...] = acc
    pltpu.sync_copy(acc_v, shared.at[s])
    plsc.subcore_barrier()

    @pl.when(s == 0)
    def _():
        pltpu.sync_copy(shared, all_v)
        tot = jnp.zeros((LANES,), jnp.float32)
        for w in range(16):
            tot = tot + all_v[w]
        acc_v[...] = tot
        pltpu.sync_copy(acc_v, out.at[c])


_sc_gather = functools.partial(
    pl.kernel,
    mesh=plsc.VectorSubcoreMesh(core_axis_name="c", subcore_axis_name="s"),
    out_type=jax.ShapeDtypeStruct((2, LANES), jnp.float32),
    scratch_types=[
        pltpu.VMEM((ROWS_PER_TILE,), jnp.int32),
        pltpu.VMEM((ROWS_PER_TILE,), jnp.int32),
        pltpu.VMEM((ROWS_PER_TILE,), jnp.float32),
        pltpu.VMEM((LANES,), jnp.float32),
        pltpu.VMEM_SHARED((16, LANES), jnp.float32),
        pltpu.VMEM((16, LANES), jnp.float32),
        pltpu.SemaphoreType.DMA,
    ],
)(_sc_body)


def kernel(output, target):
    out2d = output.reshape(N_ROWS, V)
    tgt = target.reshape(N_ROWS).astype(jnp.int32)
    tab = out2d.reshape(N_ROWS * V)
    part = _sc_gather(tab, tgt)
    a, cnt = _tc_stats(out2d, tgt.reshape(N_ROWS, 1))
    t_sum = jnp.sum(part)
    return (a[0, 0] - (CONF - SM) * t_sum) / cnt[0, 0]
